# Initial kernel scaffold; baseline (speedup 1.0000x reference)
import jax
import jax.numpy as jnp
from jax.experimental import pallas as pl


def kernel(tendency, periodicity, fc1_W, fc1_b, fc2_W, fc2_b, attn_W, attn_a, lstm_Wih, lstm_Whh, lstm_bih, lstm_bhh, out_W, out_b):
    raise NotImplementedError("write your pallas kernel here")


# speedup vs baseline: 72.0088x; 72.0088x over previous
import jax
import jax.numpy as jnp
from jax.experimental import pallas as pl

V = 128
HID = 64
M = 8
N_P = 4
B = 16
QQ = 8
RR = 8


def _leaky(x):
    return jnp.where(x >= 0, x, 0.2 * x)


def _iota2(shape, dim):
    return jax.lax.broadcasted_iota(jnp.int32, shape, dim)


def _body(tend_ref, per_ref, w1r_ref, fc1b_ref, fc2W_ref, fc2b_ref,
          attnW_ref, attna_ref, Wih_ref, Whh_ref, bih_ref, bhh_ref,
          outW_ref, outb_ref, out_ref):
    f32 = jnp.float32
    tend = tend_ref[...]
    per = per_ref[...]
    w1r = w1r_ref[...]
    fc1b = fc1b_ref[...]
    fc2W = fc2W_ref[...]
    fc2b = fc2b_ref[...]
    attnW = attnW_ref[...]
    attna = attna_ref[...]
    Wih = Wih_ref[...]
    Whh = Whh_ref[...]
    bih = bih_ref[...]
    bhh = bhh_ref[...]
    outW = outW_ref[...]
    outb = outb_ref[...]

    def rowmat(x, W):
        return jax.lax.dot_general(x, W, (((1,), (1,)), ((), ())),
                                   preferred_element_type=f32)

    u = rowmat(w1r, fc2W)
    c = rowmat(fc1b, fc2W) + fc2b

    p = rowmat(u, attnW)
    q = rowmat(c, attnW)
    a1 = attna[:, :HID]
    a2 = attna[:, HID:]
    A = jnp.sum(a1 * p)
    Ar = jnp.sum(a2 * p)
    C = jnp.sum((a1 + a2) * q)

    gu = rowmat(u, Wih)
    gcon = rowmat(c, Wih) + bih + bhh

    P1 = jnp.where(_iota2((V, QQ), 0) // 16 == _iota2((V, QQ), 1),
                   f32(1.0 / 16.0), f32(0.0))
    P2 = jnp.where(_iota2((V, RR), 0) % 8 == _iota2((V, RR), 1),
                   f32(1.0 / 16.0), f32(0.0))
    R1 = jnp.where(_iota2((QQ, 64), 1) // 8 == _iota2((QQ, 64), 0),
                   f32(1.0), f32(0.0))
    R2 = jnp.where(_iota2((RR, 64), 1) % 8 == _iota2((RR, 64), 0),
                   f32(1.0), f32(0.0))
    vcol = _iota2((64, V), 1)
    Xp = jnp.where(8 * (vcol // 16) + vcol % 8 == _iota2((64, V), 0),
                   f32(1.0), f32(0.0))

    def s_field(sig):
        m1 = jnp.dot(sig, P1, preferred_element_type=f32)
        m2 = jnp.dot(sig, P2, preferred_element_type=f32)
        M1e = jnp.dot(m1, R1, preferred_element_type=f32)
        M2e = jnp.dot(m2, R2, preferred_element_type=f32)
        mref = 0.5 * (M1e + M2e)
        z0 = _leaky(A * M1e + Ar * mref + C)
        z1 = _leaky(A * M2e + Ar * mref + C)
        alpha0 = jax.nn.sigmoid(z0 - z1)
        return alpha0 * M1e + (1.0 - alpha0) * M2e

    gu3 = gu.reshape(1, 1, 4 * HID)
    gc3 = gcon.reshape(1, 1, 4 * HID)

    def run_lstm(S, T):
        h = jnp.zeros((B, 64, HID), f32)
        cc = jnp.zeros((B, 64, HID), f32)
        for t in range(T):
            S_t = S[t * B:(t + 1) * B, :]
            gates = (S_t[:, :, None] * gu3 + gc3 +
                     jax.lax.dot_general(h, Whh, (((2,), (1,)), ((), ())),
                                         preferred_element_type=f32))
            i = jax.nn.sigmoid(gates[..., 0 * HID:1 * HID])
            f = jax.nn.sigmoid(gates[..., 1 * HID:2 * HID])
            g = jnp.tanh(gates[..., 2 * HID:3 * HID])
            o = jax.nn.sigmoid(gates[..., 3 * HID:4 * HID])
            cc = f * cc + i * g
            h = o * jnp.tanh(cc)
        return h

    h_t = run_lstm(s_field(tend), M)
    h_p = run_lstm(s_field(per), N_P)

    w_t = outW[0, :HID].reshape(1, 1, HID)
    w_p = outW[0, HID:].reshape(1, 1, HID)
    O = (jnp.sum(h_t * w_t, axis=2) + jnp.sum(h_p * w_p, axis=2)
         + outb[0, 0])
    out_ref[...] = jnp.dot(O, Xp, preferred_element_type=f32)


def kernel(tendency, periodicity, fc1_W, fc1_b, fc2_W, fc2_b, attn_W, attn_a,
           lstm_Wih, lstm_Whh, lstm_bih, lstm_bhh, out_W, out_b):
    f32 = jnp.float32
    tend2d = jnp.transpose(tendency, (1, 0, 2)).reshape(M * B, V)
    per2d = jnp.transpose(periodicity, (1, 0, 2)).reshape(N_P * B, V)
    args = (
        tend2d.astype(f32),
        per2d.astype(f32),
        fc1_W.reshape(1, HID).astype(f32),
        fc1_b.reshape(1, HID).astype(f32),
        fc2_W.astype(f32),
        fc2_b.reshape(1, HID).astype(f32),
        attn_W.astype(f32),
        attn_a.reshape(1, 2 * HID).astype(f32),
        lstm_Wih.astype(f32),
        lstm_Whh.astype(f32),
        lstm_bih.reshape(1, 4 * HID).astype(f32),
        lstm_bhh.reshape(1, 4 * HID).astype(f32),
        out_W.reshape(1, 2 * HID).astype(f32),
        out_b.reshape(1, 1).astype(f32),
    )
    out = pl.pallas_call(
        _body,
        out_shape=jax.ShapeDtypeStruct((B, V), f32),
    )(*args)
    return out
